# baseline (speedup 1.0000x reference)
import jax
import jax.numpy as jnp
from jax.experimental import pallas as pl
from jax.experimental.pallas import tpu as pltpu

N_BASES = 4
L = 1024
BI = 512


def _body(tab_ref, seqc_ref, seqr_ref, out_ref):
    c = pl.program_id(0)
    cm = jax.lax.rem(c, N_BASES)
    seqc = seqc_ref[...]
    seqr = seqr_ref[...]
    acc_i = jnp.zeros((BI, 1), jnp.float32)
    acc_j = jnp.zeros((1, L), jnp.float32)
    for k in range(N_BASES):
        t = tab_ref[k, cm]
        acc_i += t * (seqc == k).astype(jnp.float32)
        acc_j += t * (seqr == k).astype(jnp.float32)
    out_ref[0] = jnp.where(
        c < N_BASES,
        jnp.broadcast_to(acc_i, (BI, L)),
        jnp.broadcast_to(acc_j, (BI, L)),
    )


def kernel(sequence, base_table):
    seq_col = sequence.reshape(L, 1)
    seq_row = sequence.reshape(1, L)
    out = pl.pallas_call(
        _body,
        grid=(2 * N_BASES, L // BI),
        in_specs=[
            pl.BlockSpec(memory_space=pltpu.SMEM),
            pl.BlockSpec((BI, 1), lambda c, i: (i, 0)),
            pl.BlockSpec((1, L), lambda c, i: (0, 0)),
        ],
        out_specs=pl.BlockSpec((1, BI, L), lambda c, i: (c, i, 0)),
        out_shape=jax.ShapeDtypeStruct((2 * N_BASES, L, L), jnp.float32),
    )(base_table, seq_col, seq_row)
    return out[None]
